# Initial kernel scaffold; baseline (speedup 1.0000x reference)
import jax
import jax.numpy as jnp
from jax.experimental import pallas as pl


def kernel(boxes, scores, labels, features, obj2target, W1, b1, W2, b2, Wvp, bvp, alpha1, logit_scale):
    raise NotImplementedError("write your pallas kernel here")


# speedup vs baseline: 43.8949x; 43.8949x over previous
import numpy as np
import jax
import jax.numpy as jnp
from jax import lax
from jax.experimental import pallas as pl

B = 8
N = 15
N_H = 5
C = 1024
H = 25
W = 25
NUM_CLASSES = 600
NUM_OBJ = 80
POOL = 7
SCALE = 1.0 / 32.0


def _pair_idx():
    xs, ys = np.meshgrid(np.arange(N), np.arange(N), indexing="ij")
    m = (xs != ys) & (xs < N_H)
    return xs[m].astype(np.int32), ys[m].astype(np.int32)


_XK, _YK = _pair_idx()
P = int(_XK.shape[0])


def _tc_body(coords_ref, labels_ref, feat_ref, o2t_ref, w1_ref, b1_ref,
             w2_ref, b2_ref, wvp_ref, bvp_ref, scal_ref,
             logits_ref, prior_ref):
    crd = coords_ref[0]
    lt = jnp.minimum(crd[0:2], crd[4:6])
    rb = jnp.maximum(crd[2:4], crd[6:8])
    gx1 = lt[0:1] * SCALE - 0.5
    gy1 = lt[1:2] * SCALE - 0.5
    gx2 = rb[0:1] * SCALE - 0.5
    gy2 = rb[1:2] * SCALE - 0.5
    offi = lax.broadcasted_iota(jnp.int32, (POOL, 1), 0)
    off = (offi.astype(jnp.float32) + 0.5) / POOL
    px = gx1 + off * (gx2 - gx1)
    py = gy1 + off * (gy2 - gy1)

    def axis_weights(pv, size):
        f0 = jnp.floor(pv)
        frac = pv - f0
        i0 = jnp.clip(f0.astype(jnp.int32), 0, size - 1)
        i1 = jnp.clip(i0 + 1, 0, size - 1)
        pos = lax.broadcasted_iota(jnp.int32, (POOL, size, P), 1)
        w = (jnp.where(pos == i0[:, None, :], (1.0 - frac)[:, None, :], 0.0)
             + jnp.where(pos == i1[:, None, :], frac[:, None, :], 0.0))
        return w.sum(axis=0) * (1.0 / POOL)

    axT = axis_weights(px, W)
    ayT = axis_weights(py, H)
    mT = jnp.concatenate([ayT[y:y + 1, :] * axT for y in range(H)], axis=0)

    feat = feat_ref[0]
    ufT = lax.dot_general(feat, mT, (((1,), (0,)), ((), ())),
                          preferred_element_type=jnp.float32)

    hT = jnp.maximum(
        lax.dot_general(w1_ref[...], ufT, (((1,), (0,)), ((), ())),
                        preferred_element_type=jnp.float32) + b1_ref[...], 0.0)
    mlpT = lax.dot_general(w2_ref[...], hT, (((1,), (0,)), ((), ())),
                           preferred_element_type=jnp.float32) + b2_ref[...]
    alpha = scal_ref[0, 0]
    mixT = alpha * mlpT + (1.0 - alpha) * ufT
    inv = 1.0 / jnp.sqrt(jnp.sum(mixT * mixT, axis=0, keepdims=True))
    normT = mixT * inv
    lg = lax.dot_general(normT, wvp_ref[...], (((0,), (1,)), ((), ())),
                         preferred_element_type=jnp.float32)
    logits_ref[0] = jnp.exp(scal_ref[0, 1]) * (lg + bvp_ref[...])

    lab = labels_ref[0]
    onehot = (lax.broadcasted_iota(jnp.int32, (NUM_OBJ, P), 0) == lab)
    onehot = onehot.astype(jnp.float32)
    sh = jnp.exp(2.8 * jnp.log(crd[8:9, :]))
    so = jnp.exp(2.8 * jnp.log(crd[9:10, :]))
    ph = lax.dot_general(onehot * sh, o2t_ref[...], (((0,), (0,)), ((), ())),
                         preferred_element_type=jnp.float32)
    po = lax.dot_general(onehot * so, o2t_ref[...], (((0,), (0,)), ((), ())),
                         preferred_element_type=jnp.float32)
    prior_ref[0, 0] = ph
    prior_ref[1, 0] = po


def kernel(boxes, scores, labels, features, obj2target, W1, b1, W2, b2,
           Wvp, bvp, alpha1, logit_scale):
    xk = jnp.asarray(_XK)
    yk = jnp.asarray(_YK)
    sub = boxes[:, xk, :].transpose(0, 2, 1)
    obj = boxes[:, yk, :].transpose(0, 2, 1)
    sh = scores[:, xk][:, None, :]
    so = scores[:, yk][:, None, :]
    pad = jnp.zeros((B, 6, P), jnp.float32)
    coords = jnp.concatenate([sub, obj, sh, so, pad], axis=1)
    labels_y = labels[:, yk][:, None, :].astype(jnp.int32)
    featr = features.reshape(B, C, H * W)
    scal = jnp.stack([alpha1, logit_scale]).reshape(1, 2).astype(jnp.float32)

    logits, prior = pl.pallas_call(
        _tc_body,
        grid=(B,),
        in_specs=[
            pl.BlockSpec((1, 16, P), lambda b: (b, 0, 0)),
            pl.BlockSpec((1, 1, P), lambda b: (b, 0, 0)),
            pl.BlockSpec((1, C, H * W), lambda b: (b, 0, 0)),
            pl.BlockSpec((NUM_OBJ, NUM_CLASSES), lambda b: (0, 0)),
            pl.BlockSpec((C // 2, C), lambda b: (0, 0)),
            pl.BlockSpec((C // 2, 1), lambda b: (0, 0)),
            pl.BlockSpec((C, C // 2), lambda b: (0, 0)),
            pl.BlockSpec((C, 1), lambda b: (0, 0)),
            pl.BlockSpec((NUM_CLASSES, C), lambda b: (0, 0)),
            pl.BlockSpec((1, NUM_CLASSES), lambda b: (0, 0)),
            pl.BlockSpec((1, 2), lambda b: (0, 0)),
        ],
        out_specs=[
            pl.BlockSpec((1, P, NUM_CLASSES), lambda b: (b, 0, 0)),
            pl.BlockSpec((2, 1, P, NUM_CLASSES), lambda b: (0, b, 0, 0)),
        ],
        out_shape=[
            jax.ShapeDtypeStruct((B, P, NUM_CLASSES), jnp.float32),
            jax.ShapeDtypeStruct((2, B, P, NUM_CLASSES), jnp.float32),
        ],
    )(coords, labels_y, featr, obj2target, W1, b1.reshape(C // 2, 1),
      W2, b2.reshape(C, 1), Wvp, bvp.reshape(1, NUM_CLASSES), scal)

    return logits.reshape(B * P, NUM_CLASSES), prior
